# baseline (speedup 1.0000x reference)
import functools

import jax
import jax.numpy as jnp
from jax import lax
from jax.experimental import pallas as pl
from jax.experimental.pallas import tpu as pltpu
from jax.experimental.pallas import tpu_sc as plsc

jax.config.update('jax_enable_x64', True)

B = 16384
F = 26
NUM_BUCKETS = 10000
N = B * F
X2 = 2 * N

_info = plsc.get_sparse_core_info()
NC, NS, L = _info.num_cores, _info.num_subcores, _info.num_lanes
NW = NC * NS
CHUNK2 = X2 // NW
CHUNK_E = CHUNK2 // 2
PERIOD = 8 * F
NVEC = PERIOD // L
assert CHUNK2 % PERIOD == 0 and CHUNK2 * NW == X2 and CHUNK2 % (2 * F) == 0

def _body(x_hbm, vid_hbm, valid_hbm, x_v, vid_v, valid_v):
    wid = lax.axis_index("s") * jnp.int32(NC) + lax.axis_index("c")
    base2 = wid * jnp.int32(CHUNK2)
    base_e = wid * jnp.int32(CHUNK_E)
    pltpu.sync_copy(x_hbm.at[pl.ds(base2, CHUNK2)], x_v)

    lane = lax.iota(jnp.int32, L)
    lane_half = lax.shift_right_logical(lane, jnp.int32(1))
    even = lax.eq(lax.bitwise_and(lane, jnp.int32(1)), jnp.int32(0))
    recip = jnp.float32(1.0 / NUM_BUCKETS)
    zero_v = lane * jnp.int32(0)
    one_v = zero_v + jnp.int32(1)

    bases = []
    for v in range(NVEC):
        t = lane_half + jnp.int32((8 * v) % F)
        fld = lax.select(t >= jnp.int32(F), t - jnp.int32(F), t)
        bases.append(fld * jnp.int32(NUM_BUCKETS + 1) + jnp.int32(2))

    @plsc.parallel_loop(jnp.int32(0), jnp.int32(CHUNK2), jnp.int32(PERIOD))
    def blk(k0):
        e0 = lax.shift_right_logical(k0, jnp.int32(1))
        for v in range(NVEC):
            off = k0 + jnp.int32(v * L)
            raw = x_v[pl.ds(off, L)]
            q = (raw.astype(jnp.float32) * recip).astype(jnp.int32)
            r = raw - q * jnp.int32(NUM_BUCKETS)
            ok = raw > jnp.int32(0)
            vid_v[pl.ds(off, L)] = lax.select(ok, bases[v] + r, zero_v)
            m = lax.select(ok, one_v, zero_v)
            plsc.store_scatter(
                valid_v, [e0 + jnp.int32(8 * v) + lane_half], m, mask=even)

    pltpu.sync_copy(vid_v, vid_hbm.at[pl.ds(base2, CHUNK2)])
    pltpu.sync_copy(valid_v, valid_hbm.at[pl.ds(base_e, CHUNK_E)])


@jax.jit
def kernel(int_feats):
    xi = lax.bitcast_convert_type(int_feats, jnp.int32).reshape(X2)
    run = functools.partial(
        pl.kernel,
        mesh=plsc.VectorSubcoreMesh(core_axis_name="c", subcore_axis_name="s"),
        compiler_params=pltpu.CompilerParams(needs_layout_passes=False),
        out_type=[
            jax.ShapeDtypeStruct((X2,), jnp.int32),
            jax.ShapeDtypeStruct((N,), jnp.int32),
        ],
        scratch_types=[
            pltpu.VMEM((CHUNK2,), jnp.int32),
            pltpu.VMEM((CHUNK2,), jnp.int32),
            pltpu.VMEM((CHUNK_E,), jnp.int32),
        ],
    )(_body)
    vid_raw, valid32 = run(xi)
    vids = lax.bitcast_convert_type(vid_raw.reshape(B, F, 2), jnp.int64)
    valid = valid32.reshape(B, F).astype(jnp.bool_)
    return vids, valid


# speedup vs baseline: 4.5103x; 4.5103x over previous
import functools

import jax
import jax.numpy as jnp
from jax import lax
from jax.experimental import pallas as pl
from jax.experimental.pallas import tpu as pltpu
from jax.experimental.pallas import tpu_sc as plsc

jax.config.update('jax_enable_x64', True)

B = 16384
F = 26
NUM_BUCKETS = 10000
N = B * F

_info = plsc.get_sparse_core_info()
NC, NS, L = _info.num_cores, _info.num_subcores, _info.num_lanes
NW = NC * NS
CHUNK = N // NW
PERIOD = 8 * F
NVEC = PERIOD // L
assert CHUNK % PERIOD == 0 and CHUNK * NW == N and CHUNK % F == 0


def _body(x_hbm, vid_hbm, valid_hbm, x_v, vid_v, valid_v):
    wid = lax.axis_index("s") * jnp.int32(NC) + lax.axis_index("c")
    base = wid * jnp.int32(CHUNK)
    pltpu.sync_copy(x_hbm.at[pl.ds(base, CHUNK)], x_v)

    lane = lax.iota(jnp.int32, L)
    recip = jnp.float32(1.0 / NUM_BUCKETS)
    zero_v = lane * jnp.int32(0)
    one_v = zero_v + jnp.int32(1)

    bases = []
    for v in range(NVEC):
        t = lane + jnp.int32((16 * v) % F)
        fld = lax.select(t >= jnp.int32(F), t - jnp.int32(F), t)
        bases.append(fld * jnp.int32(NUM_BUCKETS + 1) + jnp.int32(2))

    @plsc.parallel_loop(jnp.int32(0), jnp.int32(CHUNK), jnp.int32(PERIOD))
    def blk(k0):
        for v in range(NVEC):
            off = k0 + jnp.int32(v * L)
            raw = x_v[pl.ds(off, L)]
            q = (raw.astype(jnp.float32) * recip).astype(jnp.int32)
            r = raw - q * jnp.int32(NUM_BUCKETS)
            ok = raw > jnp.int32(0)
            vid_v[pl.ds(off, L)] = lax.select(ok, bases[v] + r, zero_v)
            valid_v[pl.ds(off, L)] = lax.select(ok, one_v, zero_v)

    pltpu.sync_copy(vid_v, vid_hbm.at[pl.ds(base, CHUNK)])
    pltpu.sync_copy(valid_v, valid_hbm.at[pl.ds(base, CHUNK)])


@jax.jit
def kernel(int_feats):
    x32 = int_feats.astype(jnp.int32).reshape(N)
    run = functools.partial(
        pl.kernel,
        mesh=plsc.VectorSubcoreMesh(core_axis_name="c", subcore_axis_name="s"),
        out_type=[
            jax.ShapeDtypeStruct((N,), jnp.int32),
            jax.ShapeDtypeStruct((N,), jnp.int32),
        ],
        scratch_types=[
            pltpu.VMEM((CHUNK,), jnp.int32),
            pltpu.VMEM((CHUNK,), jnp.int32),
            pltpu.VMEM((CHUNK,), jnp.int32),
        ],
    )(_body)
    vid32, valid32 = run(x32)
    vids = vid32.astype(jnp.int64).reshape(B, F)
    valid = valid32.astype(jnp.bool_).reshape(B, F)
    return vids, valid


# speedup vs baseline: 4.5340x; 1.0053x over previous
import functools

import jax
import jax.numpy as jnp
from jax import lax
from jax.experimental import pallas as pl
from jax.experimental.pallas import tpu as pltpu
from jax.experimental.pallas import tpu_sc as plsc

jax.config.update('jax_enable_x64', True)

B = 16384
F = 26
NUM_BUCKETS = 10000
N = B * F

_info = plsc.get_sparse_core_info()
NC, NS, L = _info.num_cores, _info.num_subcores, _info.num_lanes
NW = NC * NS
CHUNK = N // NW
PERIOD = 8 * F
NVEC = PERIOD // L
assert CHUNK % PERIOD == 0 and CHUNK * NW == N and CHUNK % F == 0


def _body(x_hbm, vid_hbm, x_v, vid_v):
    wid = lax.axis_index("s") * jnp.int32(NC) + lax.axis_index("c")
    base = wid * jnp.int32(CHUNK)
    pltpu.sync_copy(x_hbm.at[pl.ds(base, CHUNK)], x_v)

    lane = lax.iota(jnp.int32, L)
    recip = jnp.float32(1.0 / NUM_BUCKETS)
    zero_v = lane * jnp.int32(0)

    bases = []
    for v in range(NVEC):
        t = lane + jnp.int32((16 * v) % F)
        fld = lax.select(t >= jnp.int32(F), t - jnp.int32(F), t)
        bases.append(fld * jnp.int32(NUM_BUCKETS + 1) + jnp.int32(2))

    @plsc.parallel_loop(jnp.int32(0), jnp.int32(CHUNK), jnp.int32(PERIOD))
    def blk(k0):
        for v in range(NVEC):
            off = k0 + jnp.int32(v * L)
            raw = x_v[pl.ds(off, L)]
            q = (raw.astype(jnp.float32) * recip).astype(jnp.int32)
            r = raw - q * jnp.int32(NUM_BUCKETS)
            ok = raw > jnp.int32(0)
            vid_v[pl.ds(off, L)] = lax.select(ok, bases[v] + r, zero_v)

    pltpu.sync_copy(vid_v, vid_hbm.at[pl.ds(base, CHUNK)])


@jax.jit
def kernel(int_feats):
    x32 = int_feats.astype(jnp.int32).reshape(N)
    run = functools.partial(
        pl.kernel,
        mesh=plsc.VectorSubcoreMesh(core_axis_name="c", subcore_axis_name="s"),
        out_type=[
            jax.ShapeDtypeStruct((N,), jnp.int32),
        ],
        scratch_types=[
            pltpu.VMEM((CHUNK,), jnp.int32),
            pltpu.VMEM((CHUNK,), jnp.int32),
        ],
    )(_body)
    (vid32,) = run(x32)
    vids = vid32.astype(jnp.int64).reshape(B, F)
    valid = (vid32 != 0).reshape(B, F)
    return vids, valid
